# Initial kernel scaffold; baseline (speedup 1.0000x reference)
import jax
import jax.numpy as jnp
from jax.experimental import pallas as pl


def kernel(output, target, one_hot):
    raise NotImplementedError("write your pallas kernel here")


# speedup vs baseline: 2.9320x; 2.9320x over previous
import functools
import math

import jax
import jax.numpy as jnp
from jax import lax
from jax.experimental import pallas as pl
from jax.experimental.pallas import tpu as pltpu
from jax.experimental.pallas import tpu_sc as plsc

V = 32000
SMOOTH_VAL = 0.1 / (V - 2)
CONFIDENCE = 0.9
C1 = SMOOTH_VAL * (V - 2) * math.log(SMOOTH_VAL) + CONFIDENCE * math.log(CONFIDENCE)

BR = 128
BC = 6400

NW = 32


def _tc_body(tgt_ref, out_ref, acc_ref):
    i = pl.program_id(0)
    j = pl.program_id(1)

    @pl.when((i == 0) & (j == 0))
    def _init():
        acc_ref[0, 0] = 0.0

    blk = out_ref[...]
    m = (tgt_ref[...] != 0).astype(jnp.float32)
    rs = jnp.sum(blk, axis=1, keepdims=True)
    part = -SMOOTH_VAL * jnp.sum(rs * m)
    extra = jnp.sum(m * (C1 + SMOOTH_VAL * blk[:, 0:1]))
    part = part + jnp.where(j == 0, extra, 0.0)
    acc_ref[0, 0] += part


def _tc_partial(out2d, tgt2d):
    n = out2d.shape[0]
    return pl.pallas_call(
        _tc_body,
        grid=(n // BR, V // BC),
        in_specs=[
            pl.BlockSpec((BR, 1), lambda i, j: (i, 0)),
            pl.BlockSpec((BR, BC), lambda i, j: (i, j)),
        ],
        out_specs=pl.BlockSpec(
            (1, 1), lambda i, j: (0, 0), memory_space=pltpu.SMEM),
        out_shape=jax.ShapeDtypeStruct((1, 1), jnp.float32),
    )(tgt2d, out2d)


def _sc_gather_partial(outflat, tgt):
    n = tgt.shape[0]
    ch = n // NW
    mesh = plsc.VectorSubcoreMesh(core_axis_name="c", subcore_axis_name="s")

    @functools.partial(
        pl.kernel,
        mesh=mesh,
        out_type=jax.ShapeDtypeStruct((NW, 16), jnp.float32),
        scratch_types=[
            pltpu.VMEM((ch,), jnp.int32),
            pltpu.VMEM((ch,), jnp.int32),
            pltpu.VMEM((ch,), jnp.float32),
            pltpu.VMEM((16,), jnp.float32),
            pltpu.SemaphoreType.DMA,
        ],
    )
    def k(outflat_hbm, tgt_hbm, o_hbm, tgt_v, idx_v, vals_v, acc_v, sem):
        wid = lax.axis_index("s") * 2 + lax.axis_index("c")
        base = wid * ch
        pltpu.sync_copy(tgt_hbm.at[pl.ds(base, ch)], tgt_v)
        for q in range(ch // 16):
            t16 = tgt_v[pl.ds(q * 16, 16)]
            rows = (base + q * 16) + lax.iota(jnp.int32, 16)
            idx_v[pl.ds(q * 16, 16)] = rows * V + t16
        pltpu.async_copy(outflat_hbm.at[idx_v], vals_v, sem).wait()
        acc = jnp.zeros((16,), jnp.float32)
        for q in range(ch // 16):
            t16 = tgt_v[pl.ds(q * 16, 16)]
            v16 = vals_v[pl.ds(q * 16, 16)]
            acc = acc + jnp.where(t16 != 0, v16, 0.0)
        acc_v[...] = acc
        pltpu.sync_copy(acc_v, o_hbm.at[wid])

    return k(outflat, tgt)


def kernel(output, target, one_hot):
    n = output.shape[0] * output.shape[1]
    out2d = output.reshape(n, V)
    tgt = target.reshape(n).astype(jnp.int32)
    acc = _tc_partial(out2d, tgt.reshape(n, 1))
    g = _sc_gather_partial(output.reshape(-1), tgt)
    return acc[0, 0] + jnp.float32(SMOOTH_VAL - CONFIDENCE) * jnp.sum(g)


# speedup vs baseline: 3.1934x; 1.0892x over previous
import functools
import math

import jax
import jax.numpy as jnp
from jax import lax
from jax.experimental import pallas as pl
from jax.experimental.pallas import tpu as pltpu
from jax.experimental.pallas import tpu_sc as plsc

V = 32000
SMOOTH_VAL = 0.1 / (V - 2)
CONFIDENCE = 0.9
C1 = SMOOTH_VAL * (V - 2) * math.log(SMOOTH_VAL) + CONFIDENCE * math.log(CONFIDENCE)

BR = 128
BC = 16000

NW = 32


def _tc_body(tgt_ref, out_ref, acc_ref):
    i = pl.program_id(0)
    j = pl.program_id(1)

    @pl.when((i == 0) & (j == 0))
    def _init():
        acc_ref[0, 0] = 0.0

    blk = out_ref[...]
    m = (tgt_ref[...] != 0).astype(jnp.float32)
    rs = jnp.sum(blk, axis=1, keepdims=True)
    part = -SMOOTH_VAL * jnp.sum(rs * m)
    extra = jnp.sum(m * (C1 + SMOOTH_VAL * blk[:, 0:1]))
    part = part + jnp.where(j == 0, extra, 0.0)
    acc_ref[0, 0] += part


def _tc_partial(out2d, tgt2d):
    n = out2d.shape[0]
    return pl.pallas_call(
        _tc_body,
        grid=(n // BR, V // BC),
        in_specs=[
            pl.BlockSpec((BR, 1), lambda i, j: (i, 0)),
            pl.BlockSpec((BR, BC), lambda i, j: (i, j)),
        ],
        out_specs=pl.BlockSpec(
            (1, 1), lambda i, j: (0, 0), memory_space=pltpu.SMEM),
        out_shape=jax.ShapeDtypeStruct((1, 1), jnp.float32),
    )(tgt2d, out2d)


def _sc_gather_partial(outflat, tgt):
    n = tgt.shape[0]
    ch = n // NW
    mesh = plsc.VectorSubcoreMesh(core_axis_name="c", subcore_axis_name="s")

    @functools.partial(
        pl.kernel,
        mesh=mesh,
        out_type=jax.ShapeDtypeStruct((NW, 16), jnp.float32),
        scratch_types=[
            pltpu.VMEM((ch,), jnp.int32),
            pltpu.VMEM((ch,), jnp.int32),
            pltpu.VMEM((ch,), jnp.float32),
            pltpu.VMEM((16,), jnp.float32),
            pltpu.SemaphoreType.DMA,
        ],
    )
    def k(outflat_hbm, tgt_hbm, o_hbm, tgt_v, idx_v, vals_v, acc_v, sem):
        wid = lax.axis_index("s") * 2 + lax.axis_index("c")
        base = wid * ch
        pltpu.sync_copy(tgt_hbm.at[pl.ds(base, ch)], tgt_v)
        for q in range(ch // 16):
            t16 = tgt_v[pl.ds(q * 16, 16)]
            rows = (base + q * 16) + lax.iota(jnp.int32, 16)
            idx_v[pl.ds(q * 16, 16)] = rows * V + t16
        pltpu.async_copy(outflat_hbm.at[idx_v], vals_v, sem).wait()
        acc = jnp.zeros((16,), jnp.float32)
        for q in range(ch // 16):
            t16 = tgt_v[pl.ds(q * 16, 16)]
            v16 = vals_v[pl.ds(q * 16, 16)]
            acc = acc + jnp.where(t16 != 0, v16, 0.0)
        acc_v[...] = acc
        pltpu.sync_copy(acc_v, o_hbm.at[wid])

    return k(outflat, tgt)


def kernel(output, target, one_hot):
    n = output.shape[0] * output.shape[1]
    out2d = output.reshape(n, V)
    tgt = target.reshape(n).astype(jnp.int32)
    acc = _tc_partial(out2d, tgt.reshape(n, 1))
    g = _sc_gather_partial(output.reshape(-1), tgt)
    return acc[0, 0] + jnp.float32(SMOOTH_VAL - CONFIDENCE) * jnp.sum(g)
